# baseline (speedup 1.0000x reference)
import functools

import jax
import jax.numpy as jnp
from jax.experimental import pallas as pl
from jax.experimental.pallas import tpu as pltpu

TOPK = 60
THRESHOLD = 0.15
NMS_TH = 0.05
NMS_TOPK = 20
MAX_REG = 35
FD = 24
N = FD * FD * FD
NC = 3 * (MAX_REG + 1)
STRIDE = 96.0 / FD
ROWS = N // 128
PAD = 64

NEG_INF = float("-inf")


def _body(cls_ref, shape_ref, off_ref, out_ref, idx_smem):
    s0 = cls_ref[0].reshape(ROWS, 128)
    lin = (jax.lax.broadcasted_iota(jnp.int32, (ROWS, 128), 0) * 128
           + jax.lax.broadcasted_iota(jnp.int32, (ROWS, 128), 1))
    lane64 = jax.lax.broadcasted_iota(jnp.int32, (1, PAD), 1)

    def topk_body(k, carry):
        s, vals = carry
        m = jnp.max(s)
        i = jnp.min(jnp.where(s >= m, lin, N))
        idx_smem[k] = i
        vals = jnp.where(lane64 == k, m, vals)
        s = jnp.where(lin == i, NEG_INF, s)
        return s, vals

    vals0 = jnp.full((1, PAD), NEG_INF, dtype=jnp.float32)
    _, vals = jax.lax.fori_loop(0, TOPK, topk_body, (s0, vals0))
    scores = jax.nn.sigmoid(vals)

    lane128 = jax.lax.broadcasted_iota(jnp.int32, (1, 128), 1)

    def gather_one(j, carry):
        dist, offs, idxs = carry
        n_j = idx_smem[j]
        r = n_j // 128
        c = n_j - r * 128
        oh = (lane128 == c).astype(jnp.float32)
        slab = shape_ref[0, :, pl.ds(r, 1), :][:, 0, :]
        col = jnp.sum(slab * oh, axis=1, keepdims=True)
        dist = jnp.where(lane64 == j, col, dist)
        oslab = off_ref[0, :, pl.ds(r, 1), :][:, 0, :]
        ocol = jnp.sum(oslab * oh, axis=1, keepdims=True)
        offs = jnp.where(lane64 == j, ocol, offs)
        idxs = jnp.where(lane64 == j, n_j, idxs)
        return dist, offs, idxs

    dist0 = jnp.zeros((NC, PAD), dtype=jnp.float32)
    offs0 = jnp.zeros((3, PAD), dtype=jnp.float32)
    idxs0 = jnp.zeros((1, PAD), dtype=jnp.int32)
    dist, offs, idxs = jax.lax.fori_loop(
        0, TOPK, gather_one, (dist0, offs0, idxs0))

    d3 = dist.reshape(3, MAX_REG + 1, PAD)
    m = jnp.max(d3, axis=1, keepdims=True)
    e = jnp.exp(d3 - m)
    p = e / jnp.sum(e, axis=1, keepdims=True)
    proj = jnp.arange(MAX_REG + 1, dtype=jnp.int32).astype(
        jnp.float32)[None, :, None]
    sizes = jnp.sum(p * proj, axis=1) * STRIDE

    az = (idxs // (FD * FD)).astype(jnp.float32)
    ay = ((idxs // FD) % FD).astype(jnp.float32)
    ax = (idxs % FD).astype(jnp.float32)
    anch = jnp.concatenate([az, ay, ax], axis=0)
    ctr = (anch + offs) * STRIDE

    half = sizes * 0.5
    lo = ctr - half
    hi = ctr + half
    vol = sizes[0:1] * sizes[1:2] * sizes[2:3]

    out_ref[0] = jnp.full((TOPK, 8), -1.0, dtype=jnp.float32)
    s_nms0 = jnp.where(scores > THRESHOLD, scores, NEG_INF)

    def ext(v2d, oh):
        return jnp.sum(jnp.where(oh, v2d, 0.0), axis=1, keepdims=True)

    def nms_body(k, s):
        m = jnp.max(s)
        ok = m > NEG_INF
        i = jnp.min(jnp.where(s >= m, lane64, PAD))
        oh = lane64 == i
        lo_i = ext(lo, oh)
        hi_i = ext(hi, oh)
        vol_i = ext(vol, oh)
        iw = jnp.clip(jnp.minimum(hi_i, hi) - jnp.maximum(lo_i, lo), 0.0)
        inter = iw[0:1] * iw[1:2] * iw[2:3]
        iou = inter / (vol_i + vol - inter + 1e-8)
        s_sup = jnp.where(jnp.logical_or(iou > NMS_TH, oh), NEG_INF, s)
        s_next = jnp.where(ok, s_sup, s)
        sc_i = ext(scores, oh)
        ctr_i = ext(ctr, oh)
        sz_i = ext(sizes, oh)
        row = jnp.concatenate(
            [jnp.ones((1, 1), jnp.float32), sc_i,
             ctr_i[0:1], ctr_i[1:2], ctr_i[2:3],
             sz_i[0:1], sz_i[1:2], sz_i[2:3]], axis=1)
        row = jnp.where(ok, row, -1.0)
        out_ref[0, pl.ds(k, 1), :] = row
        return s_next

    jax.lax.fori_loop(0, NMS_TOPK, nms_body, s_nms0)


@jax.jit
def kernel(Cls, Shape, Offset):
    B = Cls.shape[0]
    cls2 = Cls.reshape(B, 1, N)
    shape4 = Shape.reshape(B, NC, ROWS, 128)
    off4 = Offset.reshape(B, 3, ROWS, 128)
    out = pl.pallas_call(
        _body,
        grid=(B,),
        in_specs=[
            pl.BlockSpec((1, 1, N), lambda b: (b, 0, 0)),
            pl.BlockSpec((1, NC, ROWS, 128), lambda b: (b, 0, 0, 0)),
            pl.BlockSpec((1, 3, ROWS, 128), lambda b: (b, 0, 0, 0)),
        ],
        out_specs=pl.BlockSpec((1, TOPK, 8), lambda b: (b, 0, 0)),
        out_shape=jax.ShapeDtypeStruct((B, TOPK, 8), jnp.float32),
        scratch_shapes=[pltpu.SMEM((PAD,), jnp.int32)],
    )(cls2, shape4, off4)
    return out


# speedup vs baseline: 1.1080x; 1.1080x over previous
import functools

import jax
import jax.numpy as jnp
from jax import lax
from jax.experimental import pallas as pl
from jax.experimental.pallas import tpu as pltpu
from jax.experimental.pallas import tpu_sc as plsc

B = 16
FD = 24
N = FD * FD * FD
NCH = 108
TOPK = 60
PAD = 64
THRESHOLD = 0.15
NMS_TH = 0.05
NMS_TOPK = 20
STRIDE = 4.0
NBLK = N // 16
NG = NBLK // 16
NROW = NCH * PAD // 128
NEG = float("-inf")


def _worker(b, cls_hbm, shape_hbm, off_hbm, out_hbm, sv, bm, bm2, topn,
            tops, idxb, dist, oidx, offg, rowd, outv, loa, hia, vola, sem):
    i16 = jnp.arange(16, dtype=jnp.int32)
    zf = jnp.zeros((16,), jnp.float32)

    pltpu.sync_copy(cls_hbm.at[b], sv)

    for v in range(4):
        topn[pl.ds(v * 16, 16)] = jnp.zeros((16,), jnp.int32)
        tops[pl.ds(v * 16, 16)] = jnp.full((16,), NEG, jnp.float32)
        bm2[pl.ds(v * 16, 16)] = jnp.full((16,), NEG, jnp.float32)

    def bm_tail(j, _):
        bm[pl.ds(NBLK + j * 16, 16)] = jnp.full((16,), NEG, jnp.float32)
        return 0
    lax.fori_loop(0, (1024 - NBLK) // 16, bm_tail, 0)

    def bm_row(g, _):
        base = g * 256
        acc = jnp.full((16,), NEG, jnp.float32)
        for k in range(16):
            acc = jnp.maximum(acc, plsc.load_gather(sv, [base + i16 * 16 + k]))
        bm[pl.ds(g * 16, 16)] = acc
        return 0
    lax.fori_loop(0, NG, bm_row, 0)

    def bm2_row(r, _):
        acc = jnp.full((16,), NEG, jnp.float32)
        for k in range(16):
            acc = jnp.maximum(
                acc, plsc.load_gather(bm, [(r * 16 + i16) * 16 + k]))
        bm2[pl.ds(r * 16, 16)] = acc
        return 0
    lax.fori_loop(0, 4, bm2_row, 0)

    def topk_body(k, _):
        v0 = bm2[pl.ds(0, 16)]
        v1 = bm2[pl.ds(16, 16)]
        v2 = bm2[pl.ds(32, 16)]
        v3 = bm2[pl.ds(48, 16)]
        m = jnp.max(jnp.maximum(jnp.maximum(v0, v1), jnp.maximum(v2, v3)))
        c0 = jnp.min(jnp.where(v0 >= m, i16, 64))
        c1 = jnp.min(jnp.where(v1 >= m, i16 + 16, 64))
        c2 = jnp.min(jnp.where(v2 >= m, i16 + 32, 64))
        c3 = jnp.min(jnp.where(v3 >= m, i16 + 48, 64))
        i2 = jnp.minimum(jnp.minimum(c0, c1), jnp.minimum(c2, c3))
        row = bm[pl.ds(i2 * 16, 16)]
        l1 = jnp.min(jnp.where(row >= m, i16, 16))
        blk = i2 * 16 + l1
        srow = sv[pl.ds(blk * 16, 16)]
        l0 = jnp.min(jnp.where(srow >= m, i16, 16))
        n = blk * 16 + l0
        kk = jnp.full((16,), k, jnp.int32)
        lane0 = i16 == 0
        plsc.store_scatter(topn, [kk], jnp.full((16,), n, jnp.int32),
                           mask=lane0)
        plsc.store_scatter(tops, [kk], jnp.full((16,), m, jnp.float32),
                           mask=lane0)
        ns = jnp.where(i16 == l0, NEG, srow)
        sv[pl.ds(blk * 16, 16)] = ns
        nrow = jnp.where(i16 == l1, jnp.max(ns), row)
        bm[pl.ds(i2 * 16, 16)] = nrow
        nb2 = jnp.max(nrow)
        q = i2 >> 4
        l2 = i2 & 15
        vq = bm2[pl.ds(q * 16, 16)]
        bm2[pl.ds(q * 16, 16)] = jnp.where(i16 == l2, nb2, vq)
        return 0
    lax.fori_loop(0, TOPK, topk_body, 0)

    base_b = b * NCH * N

    def idx_row(j, _):
        for v in range(8):
            f = j * 128 + v * 16 + i16
            c = f >> 6
            p = f & 63
            nv = plsc.load_gather(topn, [p])
            idxb[pl.ds(j * 128 + v * 16, 16)] = base_b + c * N + nv
        return 0
    lax.fori_loop(0, NROW, idx_row, 0)

    obase = b * 3 * N
    for v in range(16):
        f = v * 16 + i16
        d = f >> 6
        p = f & 63
        nv = plsc.load_gather(topn, [p])
        ok = f < 192
        oidx[pl.ds(v * 16, 16)] = jnp.where(ok, obase + d * N + nv, 0)

    def fire(j, _):
        pltpu.async_copy(shape_hbm.at[idxb.at[pl.ds(j * 128, 128)]],
                         dist.at[pl.ds(j * 128, 128)], sem)
        return 0
    lax.fori_loop(0, NROW, fire, 0)
    for h in range(2):
        pltpu.async_copy(off_hbm.at[oidx.at[pl.ds(h * 128, 128)]],
                         offg.at[pl.ds(h * 128, 128)], sem)

    def drain(j, _):
        pltpu.make_async_copy(shape_hbm.at[idxb.at[pl.ds(j * 128, 128)]],
                              dist.at[pl.ds(j * 128, 128)], sem).wait()
        return 0
    lax.fori_loop(0, NROW, drain, 0)
    for h in range(2):
        pltpu.make_async_copy(off_hbm.at[oidx.at[pl.ds(h * 128, 128)]],
                              offg.at[pl.ds(h * 128, 128)], sem).wait()

    s_nms = []
    for v in range(4):
        n = topn[pl.ds(v * 16, 16)]
        logit = tops[pl.ds(v * 16, 16)]
        score = 1.0 / (1.0 + jnp.exp(-logit))
        az = (n // 576).astype(jnp.float32)
        rem = n - (n // 576) * 576
        ay = (rem // 24).astype(jnp.float32)
        ax = (rem - (rem // 24) * 24).astype(jnp.float32)
        ctr = []
        for d, a in enumerate((az, ay, ax)):
            off = offg[pl.ds(d * 64 + v * 16, 16)]
            ctr.append((a + off) * STRIDE)
        szs = []
        for d in range(3):
            def mx_body(kk, m):
                x = dist[pl.ds((d * 36 + kk) * 64 + v * 16, 16)]
                return jnp.maximum(m, x)
            m = lax.fori_loop(0, 36, mx_body, jnp.full((16,), NEG,
                                                       jnp.float32))

            def sm_body(kk, c):
                s, a = c
                x = dist[pl.ds((d * 36 + kk) * 64 + v * 16, 16)]
                e = jnp.exp(x - m)
                return s + e, a + e * kk.astype(jnp.float32)
            s, a = lax.fori_loop(0, 36, sm_body, (zf, zf))
            szs.append(a / s * STRIDE)
        half = [x * 0.5 for x in szs]
        lo = [c - h for c, h in zip(ctr, half)]
        hi = [c + h for c, h in zip(ctr, half)]
        vol = szs[0] * szs[1] * szs[2]
        for d in range(3):
            loa[pl.ds(d * 64 + v * 16, 16)] = lo[d]
            hia[pl.ds(d * 64 + v * 16, 16)] = hi[d]
        vola[pl.ds(v * 16, 16)] = vol
        rowbase = (v * 16 + i16) * 16
        comps = [jnp.ones((16,), jnp.float32), score,
                 ctr[0], ctr[1], ctr[2], szs[0], szs[1], szs[2],
                 lo[0], lo[1], lo[2], hi[0], hi[1], hi[2], vol]
        for p, val in enumerate(comps):
            plsc.store_scatter(rowd, [rowbase + p], val)
        s_nms.append(jnp.where(score > THRESHOLD, score, NEG))

    def pre(j, _):
        outv[pl.ds(j * 16, 16)] = jnp.full((16,), -1.0, jnp.float32)
        return 0
    lax.fori_loop(0, 30, pre, 0)

    def nms_body(k, s):
        s0, s1, s2, s3 = s
        m = jnp.max(jnp.maximum(jnp.maximum(s0, s1), jnp.maximum(s2, s3)))
        ok = m > NEG
        c0 = jnp.min(jnp.where(s0 >= m, i16, 64))
        c1 = jnp.min(jnp.where(s1 >= m, i16 + 16, 64))
        c2 = jnp.min(jnp.where(s2 >= m, i16 + 32, 64))
        c3 = jnp.min(jnp.where(s3 >= m, i16 + 48, 64))
        i = jnp.minimum(jnp.minimum(c0, c1), jnp.minimum(c2, c3))
        rowvec = rowd[pl.ds(i * 16, 16)]

        def ext(p):
            return jnp.max(jnp.where(i16 == p, rowvec, NEG))
        lo_i = [ext(8), ext(9), ext(10)]
        hi_i = [ext(11), ext(12), ext(13)]
        vol_i = ext(14)
        oks = jnp.full((16,), ok)
        out = []
        for j, sj in enumerate((s0, s1, s2, s3)):
            inter = jnp.ones((16,), jnp.float32)
            for d in range(3):
                lod = loa[pl.ds(d * 64 + j * 16, 16)]
                hid = hia[pl.ds(d * 64 + j * 16, 16)]
                iw = jnp.maximum(jnp.minimum(hi_i[d], hid)
                                 - jnp.maximum(lo_i[d], lod), 0.0)
                inter = inter * iw
            volj = vola[pl.ds(j * 16, 16)]
            iou = inter / (vol_i + volj - inter + 1e-8)
            supp = jnp.logical_or(iou > NMS_TH, (i16 + j * 16) == i)
            s_sup = jnp.where(supp, NEG, sj)
            out.append(jnp.where(oks, s_sup, sj))
        row8 = jnp.where(jnp.logical_and(i16 < 8, oks), rowvec, -1.0)
        plsc.store_scatter(outv, [k * 8 + i16], row8, mask=i16 < 8)
        return tuple(out)
    lax.fori_loop(0, NMS_TOPK, nms_body, tuple(s_nms))

    pltpu.sync_copy(outv, out_hbm.at[b])


def _sc_body(cls_hbm, shape_hbm, off_hbm, out_hbm, sv, bm, bm2, topn, tops,
             idxb, dist, oidx, offg, rowd, outv, loa, hia, vola, sem):
    wid = lax.axis_index("s") * 2 + lax.axis_index("c")

    @pl.when(wid < B)
    def _():
        _worker(wid, cls_hbm, shape_hbm, off_hbm, out_hbm, sv, bm, bm2,
                topn, tops, idxb, dist, oidx, offg, rowd, outv, loa, hia,
                vola, sem)


@jax.jit
def kernel(Cls, Shape, Offset):
    mesh = plsc.VectorSubcoreMesh(core_axis_name="c", subcore_axis_name="s")
    f = functools.partial(
        pl.kernel, mesh=mesh,
        compiler_params=pltpu.CompilerParams(needs_layout_passes=False),
        out_type=jax.ShapeDtypeStruct((B, TOPK * 8), jnp.float32),
        scratch_types=[
            pltpu.VMEM((N,), jnp.float32),
            pltpu.VMEM((1024,), jnp.float32),
            pltpu.VMEM((64,), jnp.float32),
            pltpu.VMEM((64,), jnp.int32),
            pltpu.VMEM((64,), jnp.float32),
            pltpu.VMEM((NCH * PAD,), jnp.int32),
            pltpu.VMEM((NCH * PAD,), jnp.float32),
            pltpu.VMEM((256,), jnp.int32),
            pltpu.VMEM((256,), jnp.float32),
            pltpu.VMEM((1024,), jnp.float32),
            pltpu.VMEM((TOPK * 8,), jnp.float32),
            pltpu.VMEM((192,), jnp.float32),
            pltpu.VMEM((192,), jnp.float32),
            pltpu.VMEM((64,), jnp.float32),
            pltpu.SemaphoreType.DMA,
        ],
    )(_sc_body)
    out = f(Cls.reshape(B, N), Shape.reshape(-1), Offset.reshape(-1))
    return out.reshape(B, TOPK, 8)


# speedup vs baseline: 10.4733x; 9.4525x over previous
import functools

import jax
import jax.numpy as jnp
from jax import lax
from jax.experimental import pallas as pl
from jax.experimental.pallas import tpu as pltpu
from jax.experimental.pallas import tpu_sc as plsc

B = 16
FD = 24
N = FD * FD * FD
NCH = 108
TOPK = 60
PAD = 64
THRESHOLD = 0.15
NMS_TH = 0.05
NMS_TOPK = 20
STRIDE = 4.0
NBLK = N // 16
NG = NBLK // 16
NROW = NCH * PAD // 128
NEG = float("-inf")


def _worker(b, cls_hbm, shape_hbm, off_hbm, out_hbm, sv, bm, bm2, topn,
            tops, topns, prow, dist, oidx, offg, rowd, outv, loa, hia, vola, sem):
    i16 = jnp.arange(16, dtype=jnp.int32)
    zf = jnp.zeros((16,), jnp.float32)

    pltpu.sync_copy(cls_hbm.at[b], sv)

    for v in range(4):
        topn[pl.ds(v * 16, 16)] = jnp.zeros((16,), jnp.int32)
        tops[pl.ds(v * 16, 16)] = jnp.full((16,), NEG, jnp.float32)
        bm2[pl.ds(v * 16, 16)] = jnp.full((16,), NEG, jnp.float32)

    def bm_tail(j, _):
        bm[pl.ds(NBLK + j * 16, 16)] = jnp.full((16,), NEG, jnp.float32)
        return 0
    lax.fori_loop(0, (1024 - NBLK) // 16, bm_tail, 0)

    def bm_row(g, _):
        base = g * 256
        acc = jnp.full((16,), NEG, jnp.float32)
        for k in range(16):
            acc = jnp.maximum(acc, plsc.load_gather(sv, [base + i16 * 16 + k]))
        bm[pl.ds(g * 16, 16)] = acc
        return 0
    lax.fori_loop(0, NG, bm_row, 0)

    def bm2_row(r, _):
        acc = jnp.full((16,), NEG, jnp.float32)
        for k in range(16):
            acc = jnp.maximum(
                acc, plsc.load_gather(bm, [(r * 16 + i16) * 16 + k]))
        bm2[pl.ds(r * 16, 16)] = acc
        return 0
    lax.fori_loop(0, 4, bm2_row, 0)

    def topk_body(k, _):
        v0 = bm2[pl.ds(0, 16)]
        v1 = bm2[pl.ds(16, 16)]
        v2 = bm2[pl.ds(32, 16)]
        v3 = bm2[pl.ds(48, 16)]
        m = jnp.max(jnp.maximum(jnp.maximum(v0, v1), jnp.maximum(v2, v3)))
        c0 = jnp.min(jnp.where(v0 >= m, i16, 64))
        c1 = jnp.min(jnp.where(v1 >= m, i16 + 16, 64))
        c2 = jnp.min(jnp.where(v2 >= m, i16 + 32, 64))
        c3 = jnp.min(jnp.where(v3 >= m, i16 + 48, 64))
        i2 = jnp.minimum(jnp.minimum(c0, c1), jnp.minimum(c2, c3))
        row = bm[pl.ds(i2 * 16, 16)]
        l1 = jnp.min(jnp.where(row >= m, i16, 16))
        blk = i2 * 16 + l1
        srow = sv[pl.ds(blk * 16, 16)]
        l0 = jnp.min(jnp.where(srow >= m, i16, 16))
        n = blk * 16 + l0
        kk = jnp.full((16,), k, jnp.int32)
        lane0 = i16 == 0
        plsc.store_scatter(topn, [kk], jnp.full((16,), n, jnp.int32),
                           mask=lane0)
        topns[k] = n
        plsc.store_scatter(tops, [kk], jnp.full((16,), m, jnp.float32),
                           mask=lane0)
        ns = jnp.where(i16 == l0, NEG, srow)
        sv[pl.ds(blk * 16, 16)] = ns
        nrow = jnp.where(i16 == l1, jnp.max(ns), row)
        bm[pl.ds(i2 * 16, 16)] = nrow
        nb2 = jnp.max(nrow)
        q = i2 >> 4
        l2 = i2 & 15
        vq = bm2[pl.ds(q * 16, 16)]
        bm2[pl.ds(q * 16, 16)] = jnp.where(i16 == l2, nb2, vq)
        return 0
    lax.fori_loop(0, TOPK, topk_body, 0)

    for j in range(TOPK, PAD):
        topns[j] = 0
    for v in range(4):
        nv = topn[pl.ds(v * 16, 16)]
        prow[pl.ds(v * 16, 16)] = (v * 16 + i16) * 8 + (nv & 7)

    obase = b * 3 * N
    for v in range(16):
        f = v * 16 + i16
        d = f >> 6
        p = f & 63
        nv = plsc.load_gather(topn, [p])
        ok = f < 192
        oidx[pl.ds(v * 16, 16)] = jnp.where(ok, obase + d * N + nv, 0)

    bbase = b * N

    def fire(j, _):
        r0 = pl.multiple_of(bbase + ((topns[j] >> 3) << 3), 8)
        pltpu.async_copy(shape_hbm.at[pl.ds(r0, 8), :],
                         dist.at[pl.ds(j * 8, 8), :], sem)
        return 0
    lax.fori_loop(0, PAD, fire, 0)
    for h in range(2):
        pltpu.async_copy(off_hbm.at[oidx.at[pl.ds(h * 128, 128)]],
                         offg.at[pl.ds(h * 128, 128)], sem)

    def drain(j, _):
        r0 = pl.multiple_of(bbase + ((topns[j] >> 3) << 3), 8)
        pltpu.make_async_copy(shape_hbm.at[pl.ds(r0, 8), :],
                              dist.at[pl.ds(j * 8, 8), :], sem).wait()
        return 0
    lax.fori_loop(0, PAD, drain, 0)
    for h in range(2):
        pltpu.make_async_copy(off_hbm.at[oidx.at[pl.ds(h * 128, 128)]],
                              offg.at[pl.ds(h * 128, 128)], sem).wait()

    s_nms = []
    for v in range(4):
        n = topn[pl.ds(v * 16, 16)]
        logit = tops[pl.ds(v * 16, 16)]
        score = 1.0 / (1.0 + jnp.exp(-logit))
        az = (n // 576).astype(jnp.float32)
        rem = n - (n // 576) * 576
        ay = (rem // 24).astype(jnp.float32)
        ax = (rem - (rem // 24) * 24).astype(jnp.float32)
        ctr = []
        for d, a in enumerate((az, ay, ax)):
            off = offg[pl.ds(d * 64 + v * 16, 16)]
            ctr.append((a + off) * STRIDE)
        szs = []
        for d in range(3):
            pvec = prow[pl.ds(v * 16, 16)]

            def mx_body(kk, m):
                x = plsc.load_gather(
                    dist, [pvec, jnp.full((16,), d * 36 + kk, jnp.int32)])
                return jnp.maximum(m, x)
            m = lax.fori_loop(0, 36, mx_body, jnp.full((16,), NEG,
                                                       jnp.float32))

            def sm_body(kk, c):
                s, a = c
                x = plsc.load_gather(
                    dist, [pvec, jnp.full((16,), d * 36 + kk, jnp.int32)])
                e = jnp.exp(x - m)
                return s + e, a + e * kk.astype(jnp.float32)
            s, a = lax.fori_loop(0, 36, sm_body, (zf, zf))
            szs.append(a / s * STRIDE)
        half = [x * 0.5 for x in szs]
        lo = [c - h for c, h in zip(ctr, half)]
        hi = [c + h for c, h in zip(ctr, half)]
        vol = szs[0] * szs[1] * szs[2]
        for d in range(3):
            loa[pl.ds(d * 64 + v * 16, 16)] = lo[d]
            hia[pl.ds(d * 64 + v * 16, 16)] = hi[d]
        vola[pl.ds(v * 16, 16)] = vol
        rowbase = (v * 16 + i16) * 16
        comps = [jnp.ones((16,), jnp.float32), score,
                 ctr[0], ctr[1], ctr[2], szs[0], szs[1], szs[2],
                 lo[0], lo[1], lo[2], hi[0], hi[1], hi[2], vol]
        for p, val in enumerate(comps):
            plsc.store_scatter(rowd, [rowbase + p], val)
        s_nms.append(jnp.where(score > THRESHOLD, score, NEG))

    def pre(j, _):
        outv[pl.ds(j * 16, 16)] = jnp.full((16,), -1.0, jnp.float32)
        return 0
    lax.fori_loop(0, 30, pre, 0)

    def nms_body(k, s):
        s0, s1, s2, s3 = s
        m = jnp.max(jnp.maximum(jnp.maximum(s0, s1), jnp.maximum(s2, s3)))
        ok = m > NEG
        c0 = jnp.min(jnp.where(s0 >= m, i16, 64))
        c1 = jnp.min(jnp.where(s1 >= m, i16 + 16, 64))
        c2 = jnp.min(jnp.where(s2 >= m, i16 + 32, 64))
        c3 = jnp.min(jnp.where(s3 >= m, i16 + 48, 64))
        i = jnp.minimum(jnp.minimum(c0, c1), jnp.minimum(c2, c3))
        rowvec = rowd[pl.ds(i * 16, 16)]

        def ext(p):
            return jnp.max(jnp.where(i16 == p, rowvec, NEG))
        lo_i = [ext(8), ext(9), ext(10)]
        hi_i = [ext(11), ext(12), ext(13)]
        vol_i = ext(14)
        oks = jnp.full((16,), ok)
        out = []
        for j, sj in enumerate((s0, s1, s2, s3)):
            inter = jnp.ones((16,), jnp.float32)
            for d in range(3):
                lod = loa[pl.ds(d * 64 + j * 16, 16)]
                hid = hia[pl.ds(d * 64 + j * 16, 16)]
                iw = jnp.maximum(jnp.minimum(hi_i[d], hid)
                                 - jnp.maximum(lo_i[d], lod), 0.0)
                inter = inter * iw
            volj = vola[pl.ds(j * 16, 16)]
            iou = inter / (vol_i + volj - inter + 1e-8)
            supp = jnp.logical_or(iou > NMS_TH, (i16 + j * 16) == i)
            s_sup = jnp.where(supp, NEG, sj)
            out.append(jnp.where(oks, s_sup, sj))
        row8 = jnp.where(jnp.logical_and(i16 < 8, oks), rowvec, -1.0)
        plsc.store_scatter(outv, [k * 8 + i16], row8, mask=i16 < 8)
        return tuple(out)
    lax.fori_loop(0, NMS_TOPK, nms_body, tuple(s_nms))

    pltpu.sync_copy(outv, out_hbm.at[b])


def _sc_body(cls_hbm, shape_hbm, off_hbm, out_hbm, sv, bm, bm2, topn, tops,
             topns, prow, dist, oidx, offg, rowd, outv, loa, hia, vola,
             sem):
    wid = lax.axis_index("s") * 2 + lax.axis_index("c")

    @pl.when(wid < B)
    def _():
        _worker(wid, cls_hbm, shape_hbm, off_hbm, out_hbm, sv, bm, bm2,
                topn, tops, topns, prow, dist, oidx, offg, rowd, outv, loa,
                hia, vola, sem)


@jax.jit
def kernel(Cls, Shape, Offset):
    mesh = plsc.VectorSubcoreMesh(core_axis_name="c", subcore_axis_name="s")
    f = functools.partial(
        pl.kernel, mesh=mesh,
        compiler_params=pltpu.CompilerParams(needs_layout_passes=False),
        out_type=jax.ShapeDtypeStruct((B, TOPK * 8), jnp.float32),
        scratch_types=[
            pltpu.VMEM((N,), jnp.float32),
            pltpu.VMEM((1024,), jnp.float32),
            pltpu.VMEM((64,), jnp.float32),
            pltpu.VMEM((64,), jnp.int32),
            pltpu.VMEM((64,), jnp.float32),
            pltpu.SMEM((PAD,), jnp.int32),
            pltpu.VMEM((PAD,), jnp.int32),
            pltpu.VMEM((PAD * 8, NCH), jnp.float32),
            pltpu.VMEM((256,), jnp.int32),
            pltpu.VMEM((256,), jnp.float32),
            pltpu.VMEM((1024,), jnp.float32),
            pltpu.VMEM((TOPK * 8,), jnp.float32),
            pltpu.VMEM((192,), jnp.float32),
            pltpu.VMEM((192,), jnp.float32),
            pltpu.VMEM((64,), jnp.float32),
            pltpu.SemaphoreType.DMA,
        ],
    )(_sc_body)
    out = f(Cls.reshape(B, N),
            Shape.transpose(0, 2, 3, 4, 1).reshape(B * N, NCH),
            Offset.reshape(-1))
    return out.reshape(B, TOPK, 8)
